# baseline (speedup 1.0000x reference)
import jax
import jax.numpy as jnp
from jax.experimental import pallas as pl
from jax.experimental.pallas import tpu as pltpu

_BATCH = 1024
_N_UNITS = 100000
_LATENT = 32

_NB = 1
_BB = _BATCH // _NB
_KB = 1024
_NK = -(-_N_UNITS // _KB)
_REM = _N_UNITS - (_NK - 1) * _KB


def _mm_kernel(mt_ref, xt_ref, o_ref):
    k = pl.program_id(1)

    @pl.when(k == 0)
    def _init():
        o_ref[...] = jnp.zeros_like(o_ref)

    @pl.when(k < _NK - 1)
    def _body():
        o_ref[...] += jnp.dot(mt_ref[...], xt_ref[...],
                              preferred_element_type=jnp.float32)

    @pl.when(k == _NK - 1)
    def _tail():
        col = jax.lax.broadcasted_iota(jnp.int32, (_LATENT, _KB), 1)
        mv = jnp.where(col < _REM, mt_ref[...], 0.0)
        row = jax.lax.broadcasted_iota(jnp.int32, (_KB, _BB), 0)
        xv = jnp.where(row < _REM, xt_ref[...], 0.0)
        o_ref[...] += jnp.dot(mv, xv, preferred_element_type=jnp.float32)


def kernel(x, m):
    xt = x.T
    mt = m.T
    out_t = pl.pallas_call(
        _mm_kernel,
        grid=(_NB, _NK),
        in_specs=[
            pl.BlockSpec((_LATENT, _KB), lambda i, k: (0, k)),
            pl.BlockSpec((_KB, _BB), lambda i, k: (k, i)),
        ],
        out_specs=pl.BlockSpec((_LATENT, _BB), lambda i, k: (0, i)),
        out_shape=jax.ShapeDtypeStruct((_LATENT, _BATCH), jnp.float32),
        compiler_params=pltpu.CompilerParams(
            dimension_semantics=("parallel", "arbitrary"),
        ),
    )(mt, xt)
    return out_t.T


# speedup vs baseline: 1.1329x; 1.1329x over previous
import jax
import jax.numpy as jnp
from jax.experimental import pallas as pl
from jax.experimental.pallas import tpu as pltpu

_BATCH = 1024
_N_UNITS = 100000
_LATENT = 32

_KB = 2048
_NK = -(-_N_UNITS // _KB)
_REM = _N_UNITS - (_NK - 1) * _KB
_P = 7
_NKP = _NK // _P


def _mm_kernel(mt_ref, xt_ref, o_ref):
    p = pl.program_id(0)
    kk = pl.program_id(1)
    k = p * _NKP + kk

    @pl.when(kk == 0)
    def _init():
        o_ref[...] = jnp.zeros_like(o_ref)

    @pl.when(k < _NK - 1)
    def _body():
        o_ref[0] += jnp.dot(mt_ref[...], xt_ref[...],
                            preferred_element_type=jnp.float32)

    @pl.when(k == _NK - 1)
    def _tail():
        col = jax.lax.broadcasted_iota(jnp.int32, (_LATENT, _KB), 1)
        mv = jnp.where(col < _REM, mt_ref[...], 0.0)
        row = jax.lax.broadcasted_iota(jnp.int32, (_KB, _BATCH), 0)
        xv = jnp.where(row < _REM, xt_ref[...], 0.0)
        o_ref[0] += jnp.dot(mv, xv, preferred_element_type=jnp.float32)


def kernel(x, m):
    xt = x.T
    mt = m.T
    partials = pl.pallas_call(
        _mm_kernel,
        grid=(_P, _NKP),
        in_specs=[
            pl.BlockSpec((_LATENT, _KB), lambda p, kk: (0, p * _NKP + kk)),
            pl.BlockSpec((_KB, _BATCH), lambda p, kk: (p * _NKP + kk, 0)),
        ],
        out_specs=pl.BlockSpec((1, _LATENT, _BATCH), lambda p, kk: (p, 0, 0)),
        out_shape=jax.ShapeDtypeStruct((_P, _LATENT, _BATCH), jnp.float32),
        compiler_params=pltpu.CompilerParams(
            dimension_semantics=("parallel", "arbitrary"),
        ),
    )(mt, xt)
    return partials.sum(axis=0).T


# speedup vs baseline: 1.1517x; 1.0166x over previous
import jax
import jax.numpy as jnp
from jax.experimental import pallas as pl
from jax.experimental.pallas import tpu as pltpu

_BATCH = 1024
_N_UNITS = 100000
_LATENT = 32

_KB = 2048
_NK = -(-_N_UNITS // _KB)
_REM = _N_UNITS - (_NK - 1) * _KB


def _mm_kernel(mt_ref, xt_ref, o_ref):
    k = pl.program_id(0)

    @pl.when(k == 0)
    def _init():
        o_ref[...] = jnp.zeros_like(o_ref)

    @pl.when(k < _NK - 1)
    def _body():
        o_ref[...] += jnp.dot(mt_ref[...], xt_ref[...],
                              preferred_element_type=jnp.float32)

    @pl.when(k == _NK - 1)
    def _tail():
        col = jax.lax.broadcasted_iota(jnp.int32, (_LATENT, _KB), 1)
        mv = jnp.where(col < _REM, mt_ref[...], 0.0)
        row = jax.lax.broadcasted_iota(jnp.int32, (_KB, _BATCH), 0)
        xv = jnp.where(row < _REM, xt_ref[...], 0.0)
        o_ref[...] += jnp.dot(mv, xv, preferred_element_type=jnp.float32)


def kernel(x, m):
    xt = x.T
    mt = m.T
    out_t = pl.pallas_call(
        _mm_kernel,
        grid=(_NK,),
        in_specs=[
            pl.BlockSpec((_LATENT, _KB), lambda k: (0, k)),
            pl.BlockSpec((_KB, _BATCH), lambda k: (k, 0)),
        ],
        out_specs=pl.BlockSpec((_LATENT, _BATCH), lambda k: (0, 0)),
        out_shape=jax.ShapeDtypeStruct((_LATENT, _BATCH), jnp.float32),
        compiler_params=pltpu.CompilerParams(
            dimension_semantics=("arbitrary",),
        ),
    )(mt, xt)
    return out_t.T
